# baseline (speedup 1.0000x reference)
import functools

import jax
import jax.numpy as jnp
from jax.experimental import pallas as pl
from jax.experimental.pallas import tpu as pltpu

_TOP_K = 32


def _body(q_ref, kt_ref, v_ref, ret_ref, out_ref, work_ref, r_ref, z_ref, *,
          scale, top_k):
    s = jax.lax.dot_general(
        q_ref[...] * scale, kt_ref[...], (((1,), (1,)), ((), ())),
        preferred_element_type=jnp.float32)
    work_ref[...] = s

    cw = r_ref.shape[1] // 2
    m_full = work_ref.shape[1]
    m1 = s[:, 0:cw]
    m2 = jnp.full_like(m1, -jnp.inf)
    for j in range(1, m_full // cw):
        b = s[:, j * cw:(j + 1) * cw]
        m2 = jnp.maximum(m2, jnp.minimum(m1, b))
        m1 = jnp.maximum(m1, b)
    r_ref[:, 0:cw] = m1
    r_ref[:, cw:2 * cw] = m2

    v0 = jnp.max(m1, axis=1, keepdims=True)

    def _step(_, m):
        rr = r_ref[...]
        return jnp.max(jnp.where(rr < m, rr, -jnp.inf), axis=1, keepdims=True)

    r32 = jax.lax.fori_loop(0, top_k - 1, _step, v0)

    kf = float(top_k)

    def _epass(t):
        sw = work_ref[...]
        e = jnp.where(sw >= t, jnp.exp(sw - v0), 0.0)
        out_ref[...] = e
        gt = sw > t
        z = jnp.sum(e, axis=1, keepdims=True)
        cnt = jnp.sum(jnp.where(gt, 1.0, 0.0), axis=1, keepdims=True)
        nxt = jnp.min(jnp.where(gt, sw, jnp.inf), axis=1, keepdims=True)
        return z, cnt, nxt

    z0, c0, n0 = _epass(r32)
    z_ref[...] = z0

    def _cond(carry):
        t, cnt, nxt = carry
        return jnp.max(cnt) >= kf

    def _rbody(carry):
        t, cnt, nxt = carry
        t = jnp.where(cnt >= kf, nxt, t)
        sw = work_ref[...]
        gt = sw > t
        cnt = jnp.sum(jnp.where(gt, 1.0, 0.0), axis=1, keepdims=True)
        nxt = jnp.min(jnp.where(gt, sw, jnp.inf), axis=1, keepdims=True)
        return t, cnt, nxt

    @pl.when(jnp.max(c0) >= kf)
    def _():
        t, _, _ = jax.lax.while_loop(_cond, _rbody, (r32, c0, n0))
        z, _, _ = _epass(t)
        z_ref[...] = z

    w = out_ref[...] * (1.0 / z_ref[...])
    out_ref[...] = w
    ret_ref[...] = jnp.dot(w, v_ref[...], preferred_element_type=jnp.float32)


def _impl(queries, mem_keys, mem_values, interpret):
    B, n_q, key_dim = queries.shape
    M, value_dim = mem_values.shape
    N = B * n_q
    tq = 256 if N % 256 == 0 else n_q
    rw = max(2 * (M // 16), 2 * min(_TOP_K, M))
    scale = 1.0 / (key_dim ** 0.5)

    qf = queries.reshape(N, key_dim)

    ret, full = pl.pallas_call(
        functools.partial(_body, scale=scale, top_k=min(_TOP_K, M)),
        grid=(N // tq,),
        in_specs=[
            pl.BlockSpec((tq, key_dim), lambda i: (i, 0)),
            pl.BlockSpec((M, key_dim), lambda i: (0, 0)),
            pl.BlockSpec((M, value_dim), lambda i: (0, 0)),
        ],
        out_specs=[
            pl.BlockSpec((tq, value_dim), lambda i: (i, 0)),
            pl.BlockSpec((tq, M), lambda i: (i, 0)),
        ],
        out_shape=[
            jax.ShapeDtypeStruct((N, value_dim), jnp.float32),
            jax.ShapeDtypeStruct((N, M), jnp.float32),
        ],
        scratch_shapes=[
            pltpu.VMEM((tq, M), jnp.float32),
            pltpu.VMEM((tq, rw), jnp.float32),
            pltpu.VMEM((tq, 1), jnp.float32),
        ],
        compiler_params=pltpu.CompilerParams(
            vmem_limit_bytes=100 * 1024 * 1024),
        interpret=interpret,
    )(qf, mem_keys, mem_values)

    return ret.reshape(B, n_q, value_dim), full.reshape(B, n_q, M)


def kernel(queries, mem_keys, mem_values):
    return _impl(queries, mem_keys, mem_values, interpret=False)


# speedup vs baseline: 1.1605x; 1.1605x over previous
import functools

import jax
import jax.numpy as jnp
from jax.experimental import pallas as pl
from jax.experimental.pallas import tpu as pltpu

_TOP_K = 32


def _body(q_ref, kt_ref, v_ref, ret_ref, out_ref, work_ref, r_ref, *,
          scale, top_k):
    s = jax.lax.dot_general(
        q_ref[...] * scale, kt_ref[...], (((1,), (1,)), ((), ())),
        preferred_element_type=jnp.float32)
    work_ref[...] = s

    cw = r_ref.shape[1] // 2
    m_full = work_ref.shape[1]
    m1 = s[:, 0:cw]
    m2 = jnp.full_like(m1, -jnp.inf)
    for j in range(1, m_full // cw):
        b = s[:, j * cw:(j + 1) * cw]
        m2 = jnp.maximum(m2, jnp.minimum(m1, b))
        m1 = jnp.maximum(m1, b)
    r_ref[:, 0:cw] = m1
    r_ref[:, cw:2 * cw] = m2

    v0 = jnp.max(m1, axis=1, keepdims=True)

    def _step(_, m):
        rr = r_ref[...]
        return jnp.max(jnp.where(rr < m, rr, -jnp.inf), axis=1, keepdims=True)

    r32 = jax.lax.fori_loop(0, top_k - 1, _step, v0)

    kf = float(top_k)

    def _probe(t):
        sw = work_ref[...]
        gt = sw > t
        cnt = jnp.sum(jnp.where(gt, 1.0, 0.0), axis=1, keepdims=True)
        nxt = jnp.min(jnp.where(gt, sw, jnp.inf), axis=1, keepdims=True)
        return cnt, nxt

    def _cond(carry):
        t, cnt, nxt = carry
        return jnp.max(cnt) >= kf

    def _rbody(carry):
        t, cnt, nxt = carry
        t = jnp.where(cnt >= kf, nxt, t)
        cnt, nxt = _probe(t)
        return t, cnt, nxt

    c0, n0 = _probe(r32)
    t, _, _ = jax.lax.while_loop(_cond, _rbody, (r32, c0, n0))

    s = work_ref[...]
    e = jnp.where(s >= t, jnp.exp(s - v0), 0.0)
    z = jnp.sum(e, axis=1, keepdims=True)
    w = e * (1.0 / z)
    out_ref[...] = w
    ret_ref[...] = jnp.dot(w, v_ref[...], preferred_element_type=jnp.float32)


def _impl(queries, mem_keys, mem_values, interpret):
    B, n_q, key_dim = queries.shape
    M, value_dim = mem_values.shape
    N = B * n_q
    tq = 256 if N % 256 == 0 else n_q
    rw = max(2 * (M // 16), 2 * min(_TOP_K, M))
    scale = 1.0 / (key_dim ** 0.5)

    qf = queries.reshape(N, key_dim)

    ret, full = pl.pallas_call(
        functools.partial(_body, scale=scale, top_k=min(_TOP_K, M)),
        grid=(N // tq,),
        in_specs=[
            pl.BlockSpec((tq, key_dim), lambda i: (i, 0)),
            pl.BlockSpec((M, key_dim), lambda i: (0, 0)),
            pl.BlockSpec((M, value_dim), lambda i: (0, 0)),
        ],
        out_specs=[
            pl.BlockSpec((tq, value_dim), lambda i: (i, 0)),
            pl.BlockSpec((tq, M), lambda i: (i, 0)),
        ],
        out_shape=[
            jax.ShapeDtypeStruct((N, value_dim), jnp.float32),
            jax.ShapeDtypeStruct((N, M), jnp.float32),
        ],
        scratch_shapes=[
            pltpu.VMEM((tq, M), jnp.float32),
            pltpu.VMEM((tq, rw), jnp.float32),
        ],
        interpret=interpret,
    )(qf, mem_keys, mem_values)

    return ret.reshape(B, n_q, value_dim), full.reshape(B, n_q, M)


def kernel(queries, mem_keys, mem_values):
    return _impl(queries, mem_keys, mem_values, interpret=False)


# speedup vs baseline: 1.3853x; 1.1937x over previous
import functools

import jax
import jax.numpy as jnp
from jax.experimental import pallas as pl
from jax.experimental.pallas import tpu as pltpu

_TOP_K = 32


def _body(q_ref, kt_ref, v_ref, ret_ref, out_ref, work_ref, r_ref, *,
          scale, top_k):
    s = jax.lax.dot_general(
        q_ref[...] * scale, kt_ref[...], (((1,), (1,)), ((), ())),
        preferred_element_type=jnp.float32)
    work_ref[...] = s

    cw = r_ref.shape[1] // 2
    m_full = work_ref.shape[1]
    m1 = s[:, 0:cw]
    m2 = jnp.full_like(m1, -jnp.inf)
    for j in range(1, m_full // cw):
        b = s[:, j * cw:(j + 1) * cw]
        m2 = jnp.maximum(m2, jnp.minimum(m1, b))
        m1 = jnp.maximum(m1, b)
    r_ref[:, 0:cw] = m1
    r_ref[:, cw:2 * cw] = m2

    v0 = jnp.max(m1, axis=1, keepdims=True)

    def _step(_, m):
        rr = r_ref[...]
        return jnp.max(jnp.where(rr < m, rr, -jnp.inf), axis=1, keepdims=True)

    r32 = jax.lax.fori_loop(0, top_k - 1, _step, v0, unroll=True)

    kf = float(top_k)

    def _probe(t):
        sw = work_ref[...]
        gt = sw > t
        cnt = jnp.sum(jnp.where(gt, 1.0, 0.0), axis=1, keepdims=True)
        nxt = jnp.min(jnp.where(gt, sw, jnp.inf), axis=1, keepdims=True)
        return cnt, nxt

    def _cond(carry):
        t, cnt, nxt = carry
        return jnp.max(cnt) >= kf

    def _rbody(carry):
        t, cnt, nxt = carry
        t = jnp.where(cnt >= kf, nxt, t)
        cnt, nxt = _probe(t)
        return t, cnt, nxt

    c0, n0 = _probe(r32)
    t, _, _ = jax.lax.while_loop(_cond, _rbody, (r32, c0, n0))

    s = work_ref[...]
    e = jnp.where(s >= t, jnp.exp(s - v0), 0.0)
    z = jnp.sum(e, axis=1, keepdims=True)
    w = e * (1.0 / z)
    out_ref[...] = w
    ret_ref[...] = jnp.dot(w, v_ref[...], preferred_element_type=jnp.float32)


def _impl(queries, mem_keys, mem_values, interpret):
    B, n_q, key_dim = queries.shape
    M, value_dim = mem_values.shape
    N = B * n_q
    tq = 256 if N % 256 == 0 else n_q
    rw = max(2 * (M // 16), 2 * min(_TOP_K, M))
    scale = 1.0 / (key_dim ** 0.5)

    qf = queries.reshape(N, key_dim)

    ret, full = pl.pallas_call(
        functools.partial(_body, scale=scale, top_k=min(_TOP_K, M)),
        grid=(N // tq,),
        in_specs=[
            pl.BlockSpec((tq, key_dim), lambda i: (i, 0)),
            pl.BlockSpec((M, key_dim), lambda i: (0, 0)),
            pl.BlockSpec((M, value_dim), lambda i: (0, 0)),
        ],
        out_specs=[
            pl.BlockSpec((tq, value_dim), lambda i: (i, 0)),
            pl.BlockSpec((tq, M), lambda i: (i, 0)),
        ],
        out_shape=[
            jax.ShapeDtypeStruct((N, value_dim), jnp.float32),
            jax.ShapeDtypeStruct((N, M), jnp.float32),
        ],
        scratch_shapes=[
            pltpu.VMEM((tq, M), jnp.float32),
            pltpu.VMEM((tq, rw), jnp.float32),
        ],
        interpret=interpret,
    )(qf, mem_keys, mem_values)

    return ret.reshape(B, n_q, value_dim), full.reshape(B, n_q, M)


def kernel(queries, mem_keys, mem_values):
    return _impl(queries, mem_keys, mem_values, interpret=False)


# speedup vs baseline: 1.3877x; 1.0017x over previous
import functools

import jax
import jax.numpy as jnp
from jax.experimental import pallas as pl
from jax.experimental.pallas import tpu as pltpu

_TOP_K = 32


def _body(q_ref, kt_ref, v_ref, ret_ref, out_ref, work_ref, r_ref, *,
          scale, top_k):
    s = jax.lax.dot_general(
        q_ref[...] * scale, kt_ref[...], (((1,), (1,)), ((), ())),
        preferred_element_type=jnp.float32)
    work_ref[...] = s

    cw = r_ref.shape[1] // 2
    m_full = work_ref.shape[1]
    m1 = s[:, 0:cw]
    m2 = jnp.full_like(m1, -jnp.inf)
    for j in range(1, m_full // cw):
        b = s[:, j * cw:(j + 1) * cw]
        m2 = jnp.maximum(m2, jnp.minimum(m1, b))
        m1 = jnp.maximum(m1, b)
    r_ref[:, 0:cw] = m1
    r_ref[:, cw:2 * cw] = m2

    v0 = jnp.max(m1, axis=1, keepdims=True)

    def _step2(_, m):
        rr = r_ref[...]
        a = jnp.where(rr < m, rr, -jnp.inf)
        m1 = jnp.max(a, axis=1, keepdims=True)
        b = jnp.where(a < m1, a, -jnp.inf)
        return jnp.max(b, axis=1, keepdims=True)

    def _step(_, m):
        rr = r_ref[...]
        return jnp.max(jnp.where(rr < m, rr, -jnp.inf), axis=1, keepdims=True)

    n2, n1 = divmod(top_k - 1, 2)
    r32 = jax.lax.fori_loop(0, n2, _step2, v0, unroll=True)
    r32 = jax.lax.fori_loop(0, n1, _step, r32, unroll=True)

    kf = float(top_k)

    def _probe(t):
        sw = work_ref[...]
        gt = sw > t
        cnt = jnp.sum(jnp.where(gt, 1.0, 0.0), axis=1, keepdims=True)
        nxt = jnp.min(jnp.where(gt, sw, jnp.inf), axis=1, keepdims=True)
        return cnt, nxt

    def _cond(carry):
        t, cnt, nxt = carry
        return jnp.max(cnt) >= kf

    def _rbody(carry):
        t, cnt, nxt = carry
        t = jnp.where(cnt >= kf, nxt, t)
        cnt, nxt = _probe(t)
        return t, cnt, nxt

    c0, n0 = _probe(r32)
    t, _, _ = jax.lax.while_loop(_cond, _rbody, (r32, c0, n0))

    s = work_ref[...]
    e = jnp.where(s >= t, jnp.exp(s - v0), 0.0)
    z = jnp.sum(e, axis=1, keepdims=True)
    w = e * (1.0 / z)
    out_ref[...] = w
    ret_ref[...] = jnp.dot(w, v_ref[...], preferred_element_type=jnp.float32)


def _impl(queries, mem_keys, mem_values, interpret):
    B, n_q, key_dim = queries.shape
    M, value_dim = mem_values.shape
    N = B * n_q
    tq = 256 if N % 256 == 0 else n_q
    rw = max(2 * (M // 16), 2 * min(_TOP_K, M))
    scale = 1.0 / (key_dim ** 0.5)

    qf = queries.reshape(N, key_dim)

    ret, full = pl.pallas_call(
        functools.partial(_body, scale=scale, top_k=min(_TOP_K, M)),
        grid=(N // tq,),
        in_specs=[
            pl.BlockSpec((tq, key_dim), lambda i: (i, 0)),
            pl.BlockSpec((M, key_dim), lambda i: (0, 0)),
            pl.BlockSpec((M, value_dim), lambda i: (0, 0)),
        ],
        out_specs=[
            pl.BlockSpec((tq, value_dim), lambda i: (i, 0)),
            pl.BlockSpec((tq, M), lambda i: (i, 0)),
        ],
        out_shape=[
            jax.ShapeDtypeStruct((N, value_dim), jnp.float32),
            jax.ShapeDtypeStruct((N, M), jnp.float32),
        ],
        scratch_shapes=[
            pltpu.VMEM((tq, M), jnp.float32),
            pltpu.VMEM((tq, rw), jnp.float32),
        ],
        interpret=interpret,
    )(qf, mem_keys, mem_values)

    return ret.reshape(B, n_q, value_dim), full.reshape(B, n_q, M)


def kernel(queries, mem_keys, mem_values):
    return _impl(queries, mem_keys, mem_values, interpret=False)
